# baseline (speedup 1.0000x reference)
import functools

import jax
import jax.numpy as jnp
from jax import lax
from jax.experimental import pallas as pl
from jax.experimental.pallas import tpu as pltpu
from jax.experimental.pallas import tpu_sc as plsc

_LAYERS = [dict(cin=0, cout=48, k=8, dil=1),
           dict(cin=48, cout=96, k=12, dil=2),
           dict(cin=96, cout=192, k=16, dil=2),
           dict(cin=192, cout=384, k=16, dil=3)]
_SAMPLES = [1024, 384, 128, 128]
_NEG_INF = -1e30
_BF = jnp.bfloat16
_F32 = jnp.float32
_SC_CORES = 2
_SC_SUBCORES = 16
_SC_TILES = _SC_CORES * _SC_SUBCORES
_CH = 128


def _elu(v):
    return jnp.where(v > 0, v, jnp.exp(jnp.minimum(v, 0.0)) - 1.0)


def _bdot(a, b, dims=None):
    if dims is None:
        dims = (((1,), (0,)), ((), ()))
    return lax.dot_general(a.astype(_BF), b.astype(_BF), dims,
                           preferred_element_type=_F32)


def _knn_body(B, K, dil, N, Mblk, pt_ref, q_ref, qsq_ref, idx_ref):
    d2s = []
    for b in range(B):
        pt = pt_ref[b]
        q = q_ref[b, 0]
        qsq = qsq_ref[b, 0]
        pt2 = pt * pt
        psq = pt2[0:1, :] + pt2[1:2, :] + pt2[2:3, :]
        qp = _bdot(q, pt)
        d2s.append((qsq + psq) - 2.0 * qp)
    neg = -jnp.concatenate(d2s, axis=0)
    R = B * Mblk
    col = lax.broadcasted_iota(jnp.int32, (R, N), 1)
    picks = []
    for t in range(K * dil):
        mx = jnp.max(neg, axis=1, keepdims=True)
        idx = jnp.min(jnp.where(neg == mx, col, N), axis=1, keepdims=True)
        if t % dil == 0:
            picks.append(idx)
        if t < K * dil - 1:
            neg = jnp.where(col == idx, _NEG_INF, neg)
    gidx = jnp.concatenate(picks, axis=1)
    gidx = gidx + (lax.broadcasted_iota(jnp.int32, (R, K), 0) // Mblk) * N
    for b in range(B):
        idx_ref[b, 0] = gidx[b * Mblk:(b + 1) * Mblk]


def _knn(p, q, i, K, dil, interpret=False):
    B, N, _ = p.shape
    M = q.shape[1]
    Mblk = 128 if M % 128 == 0 else M
    Mj = M // Mblk
    grid = (Mj,)
    body = functools.partial(_knn_body, B, K, dil, N, Mblk)
    qsq = jnp.sum(q * q, axis=2, keepdims=True)
    out = pl.pallas_call(
        body,
        grid=grid,
        in_specs=[
            pl.BlockSpec((B, 3, N), lambda j: (0, 0, 0)),
            pl.BlockSpec((B, 1, Mblk, 3), lambda j: (0, j, 0, 0)),
            pl.BlockSpec((B, 1, Mblk, 1), lambda j: (0, j, 0, 0)),
        ],
        out_specs=pl.BlockSpec((B, 1, Mblk, K), lambda j: (0, j, 0, 0)),
        out_shape=jax.ShapeDtypeStruct((B, Mj, Mblk, K), jnp.int32),
        interpret=interpret,
    )(jnp.transpose(p, (0, 2, 1)), q.reshape(B, Mj, Mblk, 3),
      qsq.reshape(B, Mj, Mblk, 1))
    return out.reshape(B, M, K)


def _sc_gather(table, idx):
    R, D = table.shape
    Btot = idx.shape[0]
    per_w = Btot // _SC_TILES
    n_ch = per_w // _CH
    mesh = plsc.VectorSubcoreMesh(core_axis_name="c", subcore_axis_name="s",
                                  num_cores=_SC_CORES,
                                  num_subcores=_SC_SUBCORES)

    @functools.partial(
        pl.kernel, mesh=mesh,
        out_type=jax.ShapeDtypeStruct((Btot, D), jnp.float32),
        compiler_params=pltpu.CompilerParams(use_tc_tiling_on_sc=False),
        scratch_types=[
            pltpu.VMEM((per_w,), jnp.int32),
            pltpu.VMEM((_CH, D), jnp.float32),
            pltpu.VMEM((_CH, D), jnp.float32),
            pltpu.SemaphoreType.DMA,
            pltpu.SemaphoreType.DMA,
            pltpu.SemaphoreType.DMA,
            pltpu.SemaphoreType.DMA,
        ],
    )
    def gather_k(table_hbm, idx_hbm, out_hbm, idx_v, rows0, rows1,
                 gs0, gs1, ss0, ss1):
        wid = lax.axis_index("s") * _SC_CORES + lax.axis_index("c")
        base = wid * per_w
        pltpu.sync_copy(idx_hbm.at[pl.ds(base, per_w)], idx_v)
        rows = (rows0, rows1)
        gsem = (gs0, gs1)
        ssem = (ss0, ss1)
        gcp = [None] * n_ch
        scp = [None] * n_ch
        gcp[0] = pltpu.async_copy(table_hbm.at[idx_v.at[pl.ds(0, _CH)]],
                                  rows0, gs0)
        for ch in range(n_ch):
            b = ch % 2
            nb = (ch + 1) % 2
            if ch + 1 < n_ch:
                if ch >= 1:
                    scp[ch - 1].wait()
                gcp[ch + 1] = pltpu.async_copy(
                    table_hbm.at[idx_v.at[pl.ds((ch + 1) * _CH, _CH)]],
                    rows[nb], gsem[nb])
            gcp[ch].wait()
            scp[ch] = pltpu.async_copy(
                rows[b], out_hbm.at[pl.ds(base + ch * _CH, _CH)], ssem[b])
        if n_ch >= 2:
            scp[n_ch - 2].wait()
        scp[n_ch - 1].wait()

    return gather_k(table, idx)


def _mlp_body(B, K, dil, cin, cd, cout, Dp, Mblk, *refs):
    (rows_ref, q_ref, w1_ref, b1_ref, w2_ref, b2_ref,
     wx_ref, bx_ref, wc_ref, out_ref) = refs
    rows = jnp.concatenate([rows_ref[b, 0] for b in range(B)], axis=0)
    q = jnp.concatenate([q_ref[b, 0] for b in range(B)], axis=0)
    R = B * Mblk
    prel = [rows[:, k * Dp:k * Dp + 3] - q for k in range(K)]
    pstack = jnp.concatenate(prel, axis=0)
    h = _elu(_bdot(pstack, w1_ref[...]) + b1_ref[...])
    h = _elu(_bdot(h, w2_ref[...]) + b2_ref[...])
    feats = [h[k * R:(k + 1) * R] for k in range(K)]
    if cin:
        feats = [jnp.concatenate(
            [feats[k], rows[:, k * Dp + 16:k * Dp + 16 + cin]], axis=1)
            for k in range(K)]
    pf = jnp.concatenate(prel, axis=1)
    xm = _bdot(pf, wx_ref[...]) + bx_ref[...]
    xmb = xm.astype(_BF).astype(_F32)
    fb = [f.astype(_BF).astype(_F32) for f in feats]
    c = cd + cin
    acc = jnp.zeros((B * Mblk, cout), _F32)
    for k in range(K):
        ft = xmb[:, k * K:k * K + 1] * fb[0]
        for j in range(1, K):
            ft = ft + xmb[:, k * K + j:k * K + j + 1] * fb[j]
        acc = acc + _bdot(ft, wc_ref[k * c:(k + 1) * c, :])
    for b in range(B):
        out_ref[b, 0] = acc[b * Mblk:(b + 1) * Mblk]


def _mlp(rows, q, params, i, L, interpret=False):
    B, M, _ = q.shape
    K, dil, cin, cout = L['k'], L['dil'], L['cin'], L['cout']
    cd = cout // 4
    c = cd + cin
    Dp = 16 + cin
    Mblk = 64 if cin >= 96 else 128
    Mj = M // Mblk
    grid = (Mj,)
    body = functools.partial(_mlp_body, B, K, dil, cin, cd, cout, Dp, Mblk)
    wspec = lambda s: pl.BlockSpec(s, lambda j: (0,) * len(s))
    rows4 = rows.reshape(B, Mj, Mblk, K * Dp)
    in_specs = [
        pl.BlockSpec((B, 1, Mblk, K * Dp), lambda j: (0, j, 0, 0)),
        pl.BlockSpec((B, 1, Mblk, 3), lambda j: (0, j, 0, 0)),
        wspec((3, cd)), wspec((1, cd)), wspec((cd, cd)), wspec((1, cd)),
        wspec((3 * K, K * K)), wspec((1, K * K)), wspec((K * c, cout)),
    ]
    ins = [rows4, q.reshape(B, Mj, Mblk, 3),
           params['c%d_w1' % i], params['c%d_b1' % i].reshape(1, cd),
           params['c%d_w2' % i], params['c%d_b2' % i].reshape(1, cd),
           params['c%d_wx' % i], params['c%d_bx' % i].reshape(1, K * K),
           params['c%d_wc' % i]]
    out = pl.pallas_call(
        body,
        grid=grid,
        in_specs=in_specs,
        out_specs=pl.BlockSpec((B, 1, Mblk, cout), lambda j: (0, j, 0, 0)),
        out_shape=jax.ShapeDtypeStruct((B, Mj, Mblk, cout), jnp.float32),
        interpret=interpret,
    )(*ins)
    return out.reshape(B, M, cout)


def _xconv(p, x, q, idx, params, i, L, interpret=False):
    B, N, _ = p.shape
    cin = L['cin']
    ppad = jnp.pad(p, ((0, 0), (0, 0), (0, 13)))
    table = (jnp.concatenate([ppad, x], axis=2) if cin else ppad)
    table = table.reshape(B * N, 16 + cin)
    rows = _sc_gather(table, idx.reshape(-1))
    return _mlp(rows, q, params, i, L, interpret=interpret)


def _head_body(B, x_ref, w1_ref, g1_ref, b1_ref, w2_ref, g2_ref, b2_ref,
               fc_ref, fcb_ref, out_ref):
    tdims = (((1,), (1,)), ((), ()))
    outs = []
    for b in range(B):
        xb = x_ref[b]
        h = _bdot(xb, w1_ref[...], tdims)
        h = jnp.maximum(h * g1_ref[...] + b1_ref[...], 0.0)
        h = _bdot(h, w2_ref[...], tdims)
        h = jnp.maximum(h * g2_ref[...] + b2_ref[...], 0.0)
        o = _bdot(h, fc_ref[...], tdims) + fcb_ref[...]
        outs.append(jnp.mean(o, axis=0, keepdims=True))
    out_ref[...] = jnp.concatenate(outs, axis=0)


def _head(x, params, interpret=False):
    B, n, C = x.shape
    ncls = params['fc_w'].shape[0]
    body = functools.partial(_head_body, B)
    ins = [x, params['mlp_w1'], params['mlp_g1'].reshape(1, -1),
           params['mlp_b1'].reshape(1, -1), params['mlp_w2'],
           params['mlp_g2'].reshape(1, -1), params['mlp_b2'].reshape(1, -1),
           params['fc_w'], params['fc_b'].reshape(1, -1)]
    return pl.pallas_call(
        body,
        out_shape=jax.ShapeDtypeStruct((B, ncls), jnp.float32),
        interpret=interpret,
    )(*ins)


def _forward_impl(p, params, interpret=False):
    pts, qs, idxs = [p], [], []
    for i, L in enumerate(_LAYERS):
        n = _SAMPLES[i]
        sidx = jax.random.permutation(jax.random.key(100 + i),
                                      pts[i].shape[1])[:n]
        q = jnp.take(pts[i], sidx, axis=1)
        qs.append(q)
        idxs.append(_knn(pts[i], q, i, L['k'], L['dil'],
                         interpret=interpret))
        pts.append(q)
    x = None
    for i, L in enumerate(_LAYERS):
        x = _xconv(pts[i], x, qs[i], idxs[i], params, i, L,
                   interpret=interpret)
    return _head(x, params, interpret=interpret)


def kernel(p, params):
    return _forward_impl(p, params)


# speedup vs baseline: 1.0204x; 1.0204x over previous
import functools

import jax
import jax.numpy as jnp
from jax import lax
from jax.experimental import pallas as pl
from jax.experimental.pallas import tpu as pltpu
from jax.experimental.pallas import tpu_sc as plsc

_LAYERS = [dict(cin=0, cout=48, k=8, dil=1),
           dict(cin=48, cout=96, k=12, dil=2),
           dict(cin=96, cout=192, k=16, dil=2),
           dict(cin=192, cout=384, k=16, dil=3)]
_SAMPLES = [1024, 384, 128, 128]
_NEG_INF = -1e30
_BF = jnp.bfloat16
_F32 = jnp.float32
_SC_CORES = 2
_SC_SUBCORES = 16
_SC_TILES = _SC_CORES * _SC_SUBCORES
_CH = 128


def _elu(v):
    return jnp.where(v > 0, v, jnp.exp(jnp.minimum(v, 0.0)) - 1.0)


def _bdot(a, b, dims=None):
    if dims is None:
        dims = (((1,), (0,)), ((), ()))
    return lax.dot_general(a.astype(_BF), b.astype(_BF), dims,
                           preferred_element_type=_F32)


def _knn_body(B, K, dil, N, Mblk, pt_ref, q_ref, qsq_ref, idx_ref):
    d2s = []
    for b in range(B):
        pt = pt_ref[b]
        q = q_ref[b, 0]
        qsq = qsq_ref[b, 0]
        pt2 = pt * pt
        psq = pt2[0:1, :] + pt2[1:2, :] + pt2[2:3, :]
        qp = _bdot(q, pt)
        d2s.append((qsq + psq) - 2.0 * qp)
    neg = -jnp.concatenate(d2s, axis=0)
    R = B * Mblk
    col = lax.broadcasted_iota(jnp.int32, (R, N), 1)
    picks = []
    for t in range(K * dil):
        mx = jnp.max(neg, axis=1, keepdims=True)
        idx = jnp.min(jnp.where(neg == mx, col, N), axis=1, keepdims=True)
        if t % dil == 0:
            picks.append(idx)
        if t < K * dil - 1:
            neg = jnp.where(col == idx, _NEG_INF, neg)
    gidx = jnp.concatenate(picks, axis=1)
    gidx = gidx + (lax.broadcasted_iota(jnp.int32, (R, K), 0) // Mblk) * N
    for b in range(B):
        idx_ref[b, 0] = gidx[b * Mblk:(b + 1) * Mblk]


def _knn(p, q, i, K, dil, interpret=False):
    B, N, _ = p.shape
    M = q.shape[1]
    Mblk = 64 if N >= 2048 else M
    Mj = M // Mblk
    grid = (Mj,)
    body = functools.partial(_knn_body, B, K, dil, N, Mblk)
    qsq = jnp.sum(q * q, axis=2, keepdims=True)
    out = pl.pallas_call(
        body,
        grid=grid,
        in_specs=[
            pl.BlockSpec((B, 3, N), lambda j: (0, 0, 0)),
            pl.BlockSpec((B, 1, Mblk, 3), lambda j: (0, j, 0, 0)),
            pl.BlockSpec((B, 1, Mblk, 1), lambda j: (0, j, 0, 0)),
        ],
        out_specs=pl.BlockSpec((B, 1, Mblk, K), lambda j: (0, j, 0, 0)),
        out_shape=jax.ShapeDtypeStruct((B, Mj, Mblk, K), jnp.int32),
        interpret=interpret,
    )(jnp.transpose(p, (0, 2, 1)), q.reshape(B, Mj, Mblk, 3),
      qsq.reshape(B, Mj, Mblk, 1))
    return out.reshape(B, M, K)


def _sc_gather(table, idx):
    R, D = table.shape
    Btot = idx.shape[0]
    per_w = Btot // _SC_TILES
    n_ch = per_w // _CH
    mesh = plsc.VectorSubcoreMesh(core_axis_name="c", subcore_axis_name="s",
                                  num_cores=_SC_CORES,
                                  num_subcores=_SC_SUBCORES)

    @functools.partial(
        pl.kernel, mesh=mesh,
        out_type=jax.ShapeDtypeStruct((Btot, D), jnp.float32),
        compiler_params=pltpu.CompilerParams(use_tc_tiling_on_sc=False),
        scratch_types=[
            pltpu.VMEM((per_w,), jnp.int32),
            pltpu.VMEM((_CH, D), jnp.float32),
            pltpu.VMEM((_CH, D), jnp.float32),
            pltpu.SemaphoreType.DMA,
            pltpu.SemaphoreType.DMA,
            pltpu.SemaphoreType.DMA,
            pltpu.SemaphoreType.DMA,
        ],
    )
    def gather_k(table_hbm, idx_hbm, out_hbm, idx_v, rows0, rows1,
                 gs0, gs1, ss0, ss1):
        wid = lax.axis_index("s") * _SC_CORES + lax.axis_index("c")
        base = wid * per_w
        pltpu.sync_copy(idx_hbm.at[pl.ds(base, per_w)], idx_v)
        rows = (rows0, rows1)
        gsem = (gs0, gs1)
        ssem = (ss0, ss1)
        gcp = [None] * n_ch
        scp = [None] * n_ch
        gcp[0] = pltpu.async_copy(table_hbm.at[idx_v.at[pl.ds(0, _CH)]],
                                  rows0, gs0)
        for ch in range(n_ch):
            b = ch % 2
            nb = (ch + 1) % 2
            if ch + 1 < n_ch:
                if ch >= 1:
                    scp[ch - 1].wait()
                gcp[ch + 1] = pltpu.async_copy(
                    table_hbm.at[idx_v.at[pl.ds((ch + 1) * _CH, _CH)]],
                    rows[nb], gsem[nb])
            gcp[ch].wait()
            scp[ch] = pltpu.async_copy(
                rows[b], out_hbm.at[pl.ds(base + ch * _CH, _CH)], ssem[b])
        if n_ch >= 2:
            scp[n_ch - 2].wait()
        scp[n_ch - 1].wait()

    return gather_k(table, idx)


def _mlp_body(B, K, dil, cin, cd, cout, Dp, Mblk, *refs):
    (rows_ref, q_ref, w1_ref, b1_ref, w2_ref, b2_ref,
     wx_ref, bx_ref, wc_ref, out_ref) = refs
    rows = jnp.concatenate([rows_ref[b, 0] for b in range(B)], axis=0)
    q = jnp.concatenate([q_ref[b, 0] for b in range(B)], axis=0)
    R = B * Mblk
    prel = [rows[:, k * Dp:k * Dp + 3] - q for k in range(K)]
    pstack = jnp.concatenate(prel, axis=0)
    h = _elu(_bdot(pstack, w1_ref[...]) + b1_ref[...])
    h = _elu(_bdot(h, w2_ref[...]) + b2_ref[...])
    feats = [h[k * R:(k + 1) * R] for k in range(K)]
    if cin:
        feats = [jnp.concatenate(
            [feats[k], rows[:, k * Dp + 16:k * Dp + 16 + cin]], axis=1)
            for k in range(K)]
    pf = jnp.concatenate(prel, axis=1)
    xm = _bdot(pf, wx_ref[...]) + bx_ref[...]
    xmb = xm.astype(_BF).astype(_F32)
    fb = [f.astype(_BF).astype(_F32) for f in feats]
    c = cd + cin
    acc = jnp.zeros((B * Mblk, cout), _F32)
    for k in range(K):
        ft = xmb[:, k * K:k * K + 1] * fb[0]
        for j in range(1, K):
            ft = ft + xmb[:, k * K + j:k * K + j + 1] * fb[j]
        acc = acc + _bdot(ft, wc_ref[k * c:(k + 1) * c, :])
    for b in range(B):
        out_ref[b, 0] = acc[b * Mblk:(b + 1) * Mblk]


def _mlp(rows, q, params, i, L, interpret=False):
    B, M, _ = q.shape
    K, dil, cin, cout = L['k'], L['dil'], L['cin'], L['cout']
    cd = cout // 4
    c = cd + cin
    Dp = 16 + cin
    Mblk = 64
    Mj = M // Mblk
    grid = (Mj,)
    body = functools.partial(_mlp_body, B, K, dil, cin, cd, cout, Dp, Mblk)
    wspec = lambda s: pl.BlockSpec(s, lambda j: (0,) * len(s))
    rows4 = rows.reshape(B, Mj, Mblk, K * Dp)
    in_specs = [
        pl.BlockSpec((B, 1, Mblk, K * Dp), lambda j: (0, j, 0, 0)),
        pl.BlockSpec((B, 1, Mblk, 3), lambda j: (0, j, 0, 0)),
        wspec((3, cd)), wspec((1, cd)), wspec((cd, cd)), wspec((1, cd)),
        wspec((3 * K, K * K)), wspec((1, K * K)), wspec((K * c, cout)),
    ]
    ins = [rows4, q.reshape(B, Mj, Mblk, 3),
           params['c%d_w1' % i], params['c%d_b1' % i].reshape(1, cd),
           params['c%d_w2' % i], params['c%d_b2' % i].reshape(1, cd),
           params['c%d_wx' % i], params['c%d_bx' % i].reshape(1, K * K),
           params['c%d_wc' % i]]
    out = pl.pallas_call(
        body,
        grid=grid,
        in_specs=in_specs,
        out_specs=pl.BlockSpec((B, 1, Mblk, cout), lambda j: (0, j, 0, 0)),
        out_shape=jax.ShapeDtypeStruct((B, Mj, Mblk, cout), jnp.float32),
        interpret=interpret,
    )(*ins)
    return out.reshape(B, M, cout)


def _xconv(p, x, q, idx, params, i, L, interpret=False):
    B, N, _ = p.shape
    cin = L['cin']
    ppad = jnp.pad(p, ((0, 0), (0, 0), (0, 13)))
    table = (jnp.concatenate([ppad, x], axis=2) if cin else ppad)
    table = table.reshape(B * N, 16 + cin)
    rows = _sc_gather(table, idx.reshape(-1))
    return _mlp(rows, q, params, i, L, interpret=interpret)


def _head_body(B, x_ref, w1_ref, g1_ref, b1_ref, w2_ref, g2_ref, b2_ref,
               fc_ref, fcb_ref, out_ref):
    tdims = (((1,), (1,)), ((), ()))
    outs = []
    for b in range(B):
        xb = x_ref[b]
        h = _bdot(xb, w1_ref[...], tdims)
        h = jnp.maximum(h * g1_ref[...] + b1_ref[...], 0.0)
        h = _bdot(h, w2_ref[...], tdims)
        h = jnp.maximum(h * g2_ref[...] + b2_ref[...], 0.0)
        o = _bdot(h, fc_ref[...], tdims) + fcb_ref[...]
        outs.append(jnp.mean(o, axis=0, keepdims=True))
    out_ref[...] = jnp.concatenate(outs, axis=0)


def _head(x, params, interpret=False):
    B, n, C = x.shape
    ncls = params['fc_w'].shape[0]
    body = functools.partial(_head_body, B)
    ins = [x, params['mlp_w1'], params['mlp_g1'].reshape(1, -1),
           params['mlp_b1'].reshape(1, -1), params['mlp_w2'],
           params['mlp_g2'].reshape(1, -1), params['mlp_b2'].reshape(1, -1),
           params['fc_w'], params['fc_b'].reshape(1, -1)]
    return pl.pallas_call(
        body,
        out_shape=jax.ShapeDtypeStruct((B, ncls), jnp.float32),
        interpret=interpret,
    )(*ins)


def _forward_impl(p, params, interpret=False):
    pts, qs, idxs = [p], [], []
    for i, L in enumerate(_LAYERS):
        n = _SAMPLES[i]
        sidx = jax.random.permutation(jax.random.key(100 + i),
                                      pts[i].shape[1])[:n]
        q = jnp.take(pts[i], sidx, axis=1)
        qs.append(q)
        idxs.append(_knn(pts[i], q, i, L['k'], L['dil'],
                         interpret=interpret))
        pts.append(q)
    x = None
    for i, L in enumerate(_LAYERS):
        x = _xconv(pts[i], x, qs[i], idxs[i], params, i, L,
                   interpret=interpret)
    return _head(x, params, interpret=interpret)


def kernel(p, params):
    return _forward_impl(p, params)


# speedup vs baseline: 1.0329x; 1.0123x over previous
import functools

import jax
import jax.numpy as jnp
from jax import lax
from jax.experimental import pallas as pl
from jax.experimental.pallas import tpu as pltpu
from jax.experimental.pallas import tpu_sc as plsc

_LAYERS = [dict(cin=0, cout=48, k=8, dil=1),
           dict(cin=48, cout=96, k=12, dil=2),
           dict(cin=96, cout=192, k=16, dil=2),
           dict(cin=192, cout=384, k=16, dil=3)]
_SAMPLES = [1024, 384, 128, 128]
_NEG_INF = -1e30
_BF = jnp.bfloat16
_F32 = jnp.float32
_SC_CORES = 2
_SC_SUBCORES = 16
_SC_TILES = _SC_CORES * _SC_SUBCORES
_CH = 128


def _elu(v):
    return jnp.where(v > 0, v, jnp.exp(jnp.minimum(v, 0.0)) - 1.0)


def _bdot(a, b, dims=None):
    if dims is None:
        dims = (((1,), (0,)), ((), ()))
    return lax.dot_general(a.astype(_BF), b.astype(_BF), dims,
                           preferred_element_type=_F32)


def _knn_body(B, K, dil, N, Mblk, pt_ref, q_ref, qsq_ref, idx_ref):
    d2s = []
    for b in range(B):
        pt = pt_ref[b]
        q = q_ref[b, 0]
        qsq = qsq_ref[b, 0]
        pt2 = pt * pt
        psq = pt2[0:1, :] + pt2[1:2, :] + pt2[2:3, :]
        qp = _bdot(q, pt)
        d2s.append((qsq + psq) - 2.0 * qp)
    neg = -jnp.concatenate(d2s, axis=0)
    R = B * Mblk
    col = lax.broadcasted_iota(jnp.int32, (R, N), 1)
    picks = []
    for t in range(K * dil):
        mx = jnp.max(neg, axis=1, keepdims=True)
        idx = jnp.min(jnp.where(neg == mx, col, N), axis=1, keepdims=True)
        if t % dil == 0:
            picks.append(idx)
        if t < K * dil - 1:
            neg = jnp.where(col == idx, _NEG_INF, neg)
    gidx = jnp.concatenate(picks, axis=1)
    gidx = gidx + (lax.broadcasted_iota(jnp.int32, (R, K), 0) // Mblk) * N
    for b in range(B):
        idx_ref[b, 0] = gidx[b * Mblk:(b + 1) * Mblk]


def _knn(p, q, i, K, dil, interpret=False):
    B, N, _ = p.shape
    M = q.shape[1]
    Mblk = 64 if N >= 2048 else (128 if M % 128 == 0 else M)
    Mj = M // Mblk
    grid = (Mj,)
    body = functools.partial(_knn_body, B, K, dil, N, Mblk)
    qsq = jnp.sum(q * q, axis=2, keepdims=True)
    out = pl.pallas_call(
        body,
        grid=grid,
        in_specs=[
            pl.BlockSpec((B, 3, N), lambda j: (0, 0, 0)),
            pl.BlockSpec((B, 1, Mblk, 3), lambda j: (0, j, 0, 0)),
            pl.BlockSpec((B, 1, Mblk, 1), lambda j: (0, j, 0, 0)),
        ],
        out_specs=pl.BlockSpec((B, 1, Mblk, K), lambda j: (0, j, 0, 0)),
        out_shape=jax.ShapeDtypeStruct((B, Mj, Mblk, K), jnp.int32),
        interpret=interpret,
    )(jnp.transpose(p, (0, 2, 1)), q.reshape(B, Mj, Mblk, 3),
      qsq.reshape(B, Mj, Mblk, 1))
    return out.reshape(B, M, K)


def _sc_gather(table, idx):
    R, D = table.shape
    Btot = idx.shape[0]
    per_w = Btot // _SC_TILES
    n_ch = per_w // _CH
    mesh = plsc.VectorSubcoreMesh(core_axis_name="c", subcore_axis_name="s",
                                  num_cores=_SC_CORES,
                                  num_subcores=_SC_SUBCORES)

    @functools.partial(
        pl.kernel, mesh=mesh,
        out_type=jax.ShapeDtypeStruct((Btot, D), jnp.float32),
        compiler_params=pltpu.CompilerParams(use_tc_tiling_on_sc=False),
        scratch_types=[
            pltpu.VMEM((per_w,), jnp.int32),
            pltpu.VMEM((_CH, D), jnp.float32),
            pltpu.VMEM((_CH, D), jnp.float32),
            pltpu.SemaphoreType.DMA,
            pltpu.SemaphoreType.DMA,
            pltpu.SemaphoreType.DMA,
            pltpu.SemaphoreType.DMA,
        ],
    )
    def gather_k(table_hbm, idx_hbm, out_hbm, idx_v, rows0, rows1,
                 gs0, gs1, ss0, ss1):
        wid = lax.axis_index("s") * _SC_CORES + lax.axis_index("c")
        base = wid * per_w
        pltpu.sync_copy(idx_hbm.at[pl.ds(base, per_w)], idx_v)
        rows = (rows0, rows1)
        gsem = (gs0, gs1)
        ssem = (ss0, ss1)
        gcp = [None] * n_ch
        scp = [None] * n_ch
        gcp[0] = pltpu.async_copy(table_hbm.at[idx_v.at[pl.ds(0, _CH)]],
                                  rows0, gs0)
        for ch in range(n_ch):
            b = ch % 2
            nb = (ch + 1) % 2
            if ch + 1 < n_ch:
                if ch >= 1:
                    scp[ch - 1].wait()
                gcp[ch + 1] = pltpu.async_copy(
                    table_hbm.at[idx_v.at[pl.ds((ch + 1) * _CH, _CH)]],
                    rows[nb], gsem[nb])
            gcp[ch].wait()
            scp[ch] = pltpu.async_copy(
                rows[b], out_hbm.at[pl.ds(base + ch * _CH, _CH)], ssem[b])
        if n_ch >= 2:
            scp[n_ch - 2].wait()
        scp[n_ch - 1].wait()

    return gather_k(table, idx)


def _mlp_body(B, K, dil, cin, cd, cout, Dp, Mblk, *refs):
    (rows_ref, q_ref, w1_ref, b1_ref, w2_ref, b2_ref,
     wx_ref, bx_ref, wc_ref, out_ref) = refs
    rows = jnp.concatenate([rows_ref[b, 0] for b in range(B)], axis=0)
    q = jnp.concatenate([q_ref[b, 0] for b in range(B)], axis=0)
    R = B * Mblk
    prel = [rows[:, k * Dp:k * Dp + 3] - q for k in range(K)]
    pstack = jnp.concatenate(prel, axis=0)
    h = _elu(_bdot(pstack, w1_ref[...]) + b1_ref[...])
    h = _elu(_bdot(h, w2_ref[...]) + b2_ref[...])
    feats = [h[k * R:(k + 1) * R] for k in range(K)]
    if cin:
        feats = [jnp.concatenate(
            [feats[k], rows[:, k * Dp + 16:k * Dp + 16 + cin]], axis=1)
            for k in range(K)]
    pf = jnp.concatenate(prel, axis=1)
    xm = _bdot(pf, wx_ref[...]) + bx_ref[...]
    xmb = xm.astype(_BF).astype(_F32)
    fb = [f.astype(_BF).astype(_F32) for f in feats]
    c = cd + cin
    acc = jnp.zeros((B * Mblk, cout), _F32)
    for k in range(K):
        ft = xmb[:, k * K:k * K + 1] * fb[0]
        for j in range(1, K):
            ft = ft + xmb[:, k * K + j:k * K + j + 1] * fb[j]
        acc = acc + _bdot(ft, wc_ref[k * c:(k + 1) * c, :])
    for b in range(B):
        out_ref[b, 0] = acc[b * Mblk:(b + 1) * Mblk]


def _mlp(rows, q, params, i, L, interpret=False):
    B, M, _ = q.shape
    K, dil, cin, cout = L['k'], L['dil'], L['cin'], L['cout']
    cd = cout // 4
    c = cd + cin
    Dp = 16 + cin
    Mblk = 64
    Mj = M // Mblk
    grid = (Mj,)
    body = functools.partial(_mlp_body, B, K, dil, cin, cd, cout, Dp, Mblk)
    wspec = lambda s: pl.BlockSpec(s, lambda j: (0,) * len(s))
    rows4 = rows.reshape(B, Mj, Mblk, K * Dp)
    in_specs = [
        pl.BlockSpec((B, 1, Mblk, K * Dp), lambda j: (0, j, 0, 0)),
        pl.BlockSpec((B, 1, Mblk, 3), lambda j: (0, j, 0, 0)),
        wspec((3, cd)), wspec((1, cd)), wspec((cd, cd)), wspec((1, cd)),
        wspec((3 * K, K * K)), wspec((1, K * K)), wspec((K * c, cout)),
    ]
    ins = [rows4, q.reshape(B, Mj, Mblk, 3),
           params['c%d_w1' % i], params['c%d_b1' % i].reshape(1, cd),
           params['c%d_w2' % i], params['c%d_b2' % i].reshape(1, cd),
           params['c%d_wx' % i], params['c%d_bx' % i].reshape(1, K * K),
           params['c%d_wc' % i]]
    out = pl.pallas_call(
        body,
        grid=grid,
        in_specs=in_specs,
        out_specs=pl.BlockSpec((B, 1, Mblk, cout), lambda j: (0, j, 0, 0)),
        out_shape=jax.ShapeDtypeStruct((B, Mj, Mblk, cout), jnp.float32),
        interpret=interpret,
    )(*ins)
    return out.reshape(B, M, cout)


def _xconv(p, x, q, idx, params, i, L, interpret=False):
    B, N, _ = p.shape
    cin = L['cin']
    ppad = jnp.pad(p, ((0, 0), (0, 0), (0, 13)))
    table = (jnp.concatenate([ppad, x], axis=2) if cin else ppad)
    table = table.reshape(B * N, 16 + cin)
    rows = _sc_gather(table, idx.reshape(-1))
    return _mlp(rows, q, params, i, L, interpret=interpret)


def _head_body(B, x_ref, w1_ref, g1_ref, b1_ref, w2_ref, g2_ref, b2_ref,
               fc_ref, fcb_ref, out_ref):
    tdims = (((1,), (1,)), ((), ()))
    outs = []
    for b in range(B):
        xb = x_ref[b]
        h = _bdot(xb, w1_ref[...], tdims)
        h = jnp.maximum(h * g1_ref[...] + b1_ref[...], 0.0)
        h = _bdot(h, w2_ref[...], tdims)
        h = jnp.maximum(h * g2_ref[...] + b2_ref[...], 0.0)
        o = _bdot(h, fc_ref[...], tdims) + fcb_ref[...]
        outs.append(jnp.mean(o, axis=0, keepdims=True))
    out_ref[...] = jnp.concatenate(outs, axis=0)


def _head(x, params, interpret=False):
    B, n, C = x.shape
    ncls = params['fc_w'].shape[0]
    body = functools.partial(_head_body, B)
    ins = [x, params['mlp_w1'], params['mlp_g1'].reshape(1, -1),
           params['mlp_b1'].reshape(1, -1), params['mlp_w2'],
           params['mlp_g2'].reshape(1, -1), params['mlp_b2'].reshape(1, -1),
           params['fc_w'], params['fc_b'].reshape(1, -1)]
    return pl.pallas_call(
        body,
        out_shape=jax.ShapeDtypeStruct((B, ncls), jnp.float32),
        interpret=interpret,
    )(*ins)


def _forward_impl(p, params, interpret=False):
    pts, qs, idxs = [p], [], []
    for i, L in enumerate(_LAYERS):
        n = _SAMPLES[i]
        sidx = jax.random.permutation(jax.random.key(100 + i),
                                      pts[i].shape[1])[:n]
        q = jnp.take(pts[i], sidx, axis=1)
        qs.append(q)
        idxs.append(_knn(pts[i], q, i, L['k'], L['dil'],
                         interpret=interpret))
        pts.append(q)
    x = None
    for i, L in enumerate(_LAYERS):
        x = _xconv(pts[i], x, qs[i], idxs[i], params, i, L,
                   interpret=interpret)
    return _head(x, params, interpret=interpret)


def kernel(p, params):
    return _forward_impl(p, params)
